# baseline (speedup 1.0000x reference)
import jax
import jax.numpy as jnp
from jax import lax
from jax.experimental import pallas as pl
from jax.experimental.pallas import tpu as pltpu
from jax.experimental.pallas import tpu_sc as plsc

_HID = 2048
_B = 2
_S = 4096
_EPS = 1e-6
_L = 16
_NC = 2
_NS = 16
_NW = _NC * _NS
_N = _B * _S
_RPW = _N // _NW
_CHUNK = 16
_NCHUNK = _RPW // _CHUNK
_SLICES = _HID // _L


def _body(ids_hbm, w_hbm, table_hbm, out_hbm, idx_v, w_v, buf_v, sem0, sem1):
    wid = lax.axis_index("s") * _NC + lax.axis_index("c")
    base = wid * _RPW
    pltpu.sync_copy(ids_hbm.at[pl.ds(base, _RPW)], idx_v)
    pltpu.sync_copy(w_hbm, w_v)
    sems = (sem0, sem1)
    iota = lax.iota(jnp.int32, _L)

    def gather_desc(c, slot):
        return pltpu.make_async_copy(
            table_hbm.at[idx_v.at[pl.ds(c * _CHUNK, _CHUNK)]],
            buf_v.at[slot],
            sems[slot],
        )

    def process(slot):
        zeros = tuple(jnp.zeros((_L,), jnp.float32) for _ in range(_CHUNK))

        @plsc.parallel_loop(0, _HID, step=_L, unroll=4, carry=zeros)
        def accs(o, acc_in):
            new = []
            for r in range(_CHUNK):
                x = buf_v[slot, r, pl.ds(o, _L)]
                new.append(acc_in[r] + x * x)
            return tuple(new)

        vecs = list(accs)
        stage = 0
        while len(vecs) > 1:
            g = 1 << stage
            sel = ((iota >> stage) & 1) == 0
            perm = jnp.bitwise_xor(iota, g)
            nxt = []
            for k in range(len(vecs) // 2):
                a, b = vecs[2 * k], vecs[2 * k + 1]
                ap = a.at[perm].get(mode="promise_in_bounds")
                bp = b.at[perm].get(mode="promise_in_bounds")
                nxt.append(jnp.where(sel, a, b) + jnp.where(sel, ap, bp))
            vecs = nxt
            stage += 1
        s = vecs[0]
        while (1 << stage) < _L:
            perm = jnp.bitwise_xor(iota, 1 << stage)
            s = s + s.at[perm].get(mode="promise_in_bounds")
            stage += 1
        vv = s * (1.0 / _HID) + _EPS
        bits = lax.bitcast_convert_type(vv, jnp.int32)
        bits = jnp.full((_L,), 0x5F3759DF, jnp.int32) - \
            lax.shift_right_logical(bits, 1)
        y = lax.bitcast_convert_type(bits, jnp.float32)
        for _ in range(4):
            y = y * (1.5 - (0.5 * vv) * (y * y))
        ysplat = [
            y.at[jnp.full((_L,), r, jnp.int32)].get(mode="promise_in_bounds")
            for r in range(_CHUNK)
        ]

        @plsc.parallel_loop(0, _HID, step=_L, unroll=4)
        def _(o):
            sl = pl.ds(o, _L)
            w = w_v[sl]
            for r in range(_CHUNK):
                buf_v[slot, r, sl] = buf_v[slot, r, sl] * ysplat[r] * w

    gather_desc(0, 0).start()
    gather_desc(1, 1).start()

    def _one(c, slot):
        gather_desc(c, slot).wait()
        process(slot)
        pltpu.sync_copy(buf_v.at[slot],
                        out_hbm.at[pl.ds(base + c * _CHUNK, _CHUNK)])

        @pl.when(c + 2 < _NCHUNK)
        def _():
            gather_desc(c + 2, slot).start()

    def pair_body(p, carry):
        for slot in range(2):
            _one(2 * p + slot, slot)
        return carry

    lax.fori_loop(0, _NCHUNK // 2, pair_body, 0)


def kernel(input_ids, embed_table, norm_weight):
    ids = input_ids.reshape(-1).astype(jnp.int32)
    mesh = plsc.VectorSubcoreMesh(core_axis_name="c", subcore_axis_name="s")
    f = pl.kernel(
        _body,
        mesh=mesh,
        out_type=jax.ShapeDtypeStruct((_N, _HID), jnp.float32),
        scratch_types=[
            pltpu.VMEM((_RPW,), jnp.int32),
            pltpu.VMEM((_HID,), jnp.float32),
            pltpu.VMEM((2, _CHUNK, _HID), jnp.float32),
            pltpu.SemaphoreType.DMA,
            pltpu.SemaphoreType.DMA,
        ],
    )
    out = f(ids, norm_weight.astype(jnp.float32), embed_table)
    return out.reshape(_B, _S, _HID)


# speedup vs baseline: 1.0903x; 1.0903x over previous
import jax
import jax.numpy as jnp
from jax import lax
from jax.experimental import pallas as pl
from jax.experimental.pallas import tpu as pltpu
from jax.experimental.pallas import tpu_sc as plsc

_HID = 2048
_B = 2
_S = 4096
_EPS = 1e-6
_L = 16
_NC = 2
_NS = 16
_NW = _NC * _NS
_N = _B * _S
_RPW = _N // _NW
_CHUNK = 16
_NCHUNK = _RPW // _CHUNK
_SLICES = _HID // _L


def _body(ids_hbm, w_hbm, table_hbm, out_hbm, idx_v, w_v, buf_v, sem0, sem1):
    wid = lax.axis_index("s") * _NC + lax.axis_index("c")
    base = wid * _RPW
    pltpu.sync_copy(ids_hbm.at[pl.ds(base, _RPW)], idx_v)
    pltpu.sync_copy(w_hbm, w_v)
    sems = (sem0, sem1)
    iota = lax.iota(jnp.int32, _L)

    def gather_desc(c, slot):
        return pltpu.make_async_copy(
            table_hbm.at[idx_v.at[pl.ds(c * _CHUNK, _CHUNK)]],
            buf_v.at[slot],
            sems[slot],
        )

    def process(slot):
        zeros = tuple(jnp.zeros((_L,), jnp.float32) for _ in range(_CHUNK))

        @plsc.parallel_loop(0, _HID, step=_L, unroll=2, carry=zeros)
        def accs(o, acc_in):
            new = []
            for r in range(_CHUNK):
                x = buf_v[slot, r, pl.ds(o, _L)]
                new.append(acc_in[r] + x * x)
            return tuple(new)

        vecs = list(accs)
        stage = 0
        while len(vecs) > 1:
            g = 1 << stage
            sel = ((iota >> stage) & 1) == 0
            perm = jnp.bitwise_xor(iota, g)
            nxt = []
            for k in range(len(vecs) // 2):
                a, b = vecs[2 * k], vecs[2 * k + 1]
                ap = a.at[perm].get(mode="promise_in_bounds")
                bp = b.at[perm].get(mode="promise_in_bounds")
                nxt.append(jnp.where(sel, a, b) + jnp.where(sel, ap, bp))
            vecs = nxt
            stage += 1
        s = vecs[0]
        while (1 << stage) < _L:
            perm = jnp.bitwise_xor(iota, 1 << stage)
            s = s + s.at[perm].get(mode="promise_in_bounds")
            stage += 1
        vv = s * (1.0 / _HID) + _EPS
        bits = lax.bitcast_convert_type(vv, jnp.int32)
        bits = jnp.full((_L,), 0x5F3759DF, jnp.int32) - \
            lax.shift_right_logical(bits, 1)
        y = lax.bitcast_convert_type(bits, jnp.float32)
        for _ in range(4):
            y = y * (1.5 - (0.5 * vv) * (y * y))
        ysplat = [
            y.at[jnp.full((_L,), r, jnp.int32)].get(mode="promise_in_bounds")
            for r in range(_CHUNK)
        ]

        @plsc.parallel_loop(0, _HID, step=_L, unroll=2)
        def _(o):
            sl = pl.ds(o, _L)
            w = w_v[sl]
            for r in range(_CHUNK):
                buf_v[slot, r, sl] = buf_v[slot, r, sl] * ysplat[r] * w

    gather_desc(0, 0).start()
    gather_desc(1, 1).start()

    def _one(c, slot):
        gather_desc(c, slot).wait()
        process(slot)
        pltpu.sync_copy(buf_v.at[slot],
                        out_hbm.at[pl.ds(base + c * _CHUNK, _CHUNK)])

        @pl.when(c + 2 < _NCHUNK)
        def _():
            gather_desc(c + 2, slot).start()

    def pair_body(p, carry):
        for slot in range(2):
            _one(2 * p + slot, slot)
        return carry

    lax.fori_loop(0, _NCHUNK // 2, pair_body, 0)


def kernel(input_ids, embed_table, norm_weight):
    ids = input_ids.reshape(-1).astype(jnp.int32)
    mesh = plsc.VectorSubcoreMesh(core_axis_name="c", subcore_axis_name="s")
    f = pl.kernel(
        _body,
        mesh=mesh,
        out_type=jax.ShapeDtypeStruct((_N, _HID), jnp.float32),
        scratch_types=[
            pltpu.VMEM((_RPW,), jnp.int32),
            pltpu.VMEM((_HID,), jnp.float32),
            pltpu.VMEM((2, _CHUNK, _HID), jnp.float32),
            pltpu.SemaphoreType.DMA,
            pltpu.SemaphoreType.DMA,
        ],
    )
    out = f(ids, norm_weight.astype(jnp.float32), embed_table)
    return out.reshape(_B, _S, _HID)


# speedup vs baseline: 1.4006x; 1.2846x over previous
import jax
import jax.numpy as jnp
from jax import lax
from jax.experimental import pallas as pl
from jax.experimental.pallas import tpu as pltpu
from jax.experimental.pallas import tpu_sc as plsc

_HID = 2048
_B = 2
_S = 4096
_EPS = 1e-6
_L = 16
_NC = 2
_NS = 16
_NW = _NC * _NS
_N = _B * _S
_RPW = _N // _NW
_CHUNK = 16
_NCHUNK = _RPW // _CHUNK
_SLICES = _HID // _L


def _body(ids_hbm, w_hbm, table_hbm, out_hbm, idx_v, w_v, buf_v, gsem, osem):
    wid = lax.axis_index("s") * _NC + lax.axis_index("c")
    base = wid * _RPW
    pltpu.sync_copy(ids_hbm.at[pl.ds(base, _RPW)], idx_v)
    pltpu.sync_copy(w_hbm, w_v)
    iota = lax.iota(jnp.int32, _L)

    def gather_desc(c, slot):
        return pltpu.make_async_copy(
            table_hbm.at[idx_v.at[pl.ds(c * _CHUNK, _CHUNK)]],
            buf_v.at[slot],
            gsem,
        )

    def store_desc(c, slot):
        return pltpu.make_async_copy(
            buf_v.at[slot],
            out_hbm.at[pl.ds(base + c * _CHUNK, _CHUNK)],
            osem,
        )

    def process(slot):
        zeros = tuple(jnp.zeros((_L,), jnp.float32) for _ in range(_CHUNK))

        @plsc.parallel_loop(0, _HID, step=_L, unroll=2, carry=zeros)
        def accs(o, acc_in):
            new = []
            for r in range(_CHUNK):
                x = buf_v[slot, r, pl.ds(o, _L)]
                new.append(acc_in[r] + x * x)
            return tuple(new)

        vecs = list(accs)
        stage = 0
        while len(vecs) > 1:
            g = 1 << stage
            sel = ((iota >> stage) & 1) == 0
            perm = jnp.bitwise_xor(iota, g)
            nxt = []
            for k in range(len(vecs) // 2):
                a, b = vecs[2 * k], vecs[2 * k + 1]
                ap = a.at[perm].get(mode="promise_in_bounds")
                bp = b.at[perm].get(mode="promise_in_bounds")
                nxt.append(jnp.where(sel, a, b) + jnp.where(sel, ap, bp))
            vecs = nxt
            stage += 1
        s = vecs[0]
        while (1 << stage) < _L:
            perm = jnp.bitwise_xor(iota, 1 << stage)
            s = s + s.at[perm].get(mode="promise_in_bounds")
            stage += 1
        vv = s * (1.0 / _HID) + _EPS
        bits = lax.bitcast_convert_type(vv, jnp.int32)
        bits = jnp.full((_L,), 0x5F3759DF, jnp.int32) - \
            lax.shift_right_logical(bits, 1)
        y = lax.bitcast_convert_type(bits, jnp.float32)
        for _ in range(4):
            y = y * (1.5 - (0.5 * vv) * (y * y))
        ysplat = [
            y.at[jnp.full((_L,), r, jnp.int32)].get(mode="promise_in_bounds")
            for r in range(_CHUNK)
        ]

        @plsc.parallel_loop(0, _HID, step=_L, unroll=2)
        def _(o):
            sl = pl.ds(o, _L)
            w = w_v[sl]
            for r in range(_CHUNK):
                buf_v[slot, r, sl] = buf_v[slot, r, sl] * ysplat[r] * w

    gather_desc(0, 0).start()

    def body(c, carry):
        slot = lax.rem(c, 3)
        nslot = lax.rem(c + 1, 3)

        @pl.when(c >= 2)
        def _():
            store_desc(c - 2, nslot).wait()

        @pl.when(c + 1 < _NCHUNK)
        def _():
            gather_desc(c + 1, nslot).start()

        gather_desc(c, slot).wait()
        process(slot)
        store_desc(c, slot).start()
        return carry

    lax.fori_loop(0, _NCHUNK, body, 0)
    store_desc(_NCHUNK - 2, (_NCHUNK - 2) % 3).wait()
    store_desc(_NCHUNK - 1, (_NCHUNK - 1) % 3).wait()


def kernel(input_ids, embed_table, norm_weight):
    ids = input_ids.reshape(-1).astype(jnp.int32)
    mesh = plsc.VectorSubcoreMesh(core_axis_name="c", subcore_axis_name="s")
    f = pl.kernel(
        _body,
        mesh=mesh,
        out_type=jax.ShapeDtypeStruct((_N, _HID), jnp.float32),
        scratch_types=[
            pltpu.VMEM((_RPW,), jnp.int32),
            pltpu.VMEM((_HID,), jnp.float32),
            pltpu.VMEM((3, _CHUNK, _HID), jnp.float32),
            pltpu.SemaphoreType.DMA,
            pltpu.SemaphoreType.DMA,
        ],
    )
    out = f(ids, norm_weight.astype(jnp.float32), embed_table)
    return out.reshape(_B, _S, _HID)
